# baseline (speedup 1.0000x reference)
import functools

import jax
import jax.numpy as jnp
from jax import lax
from jax.experimental import pallas as pl
from jax.experimental.pallas import tpu as pltpu
from jax.experimental.pallas import tpu_sc as plsc

_NC = 2
_NS = 16
_NW = _NC * _NS
_CHUNK = 128


def _round_up(a: int, b: int) -> int:
    return (a + b - 1) // b * b


@functools.lru_cache(maxsize=None)
def _make_aggregate(n, d, k, npad, kp):
    mesh = plsc.VectorSubcoreMesh(core_axis_name="c", subcore_axis_name="s")
    zrows = npad // _NS
    nzc = zrows // _CHUNK
    zrem = zrows - nzc * _CHUNK
    npass = k // kp

    @functools.partial(
        pl.kernel,
        out_type=jax.ShapeDtypeStruct((_NC, npad, d), jnp.float32),
        mesh=mesh,
        scratch_types=[
            pltpu.VMEM((kp, _CHUNK), jnp.int32),
            pltpu.VMEM((kp, _CHUNK), jnp.int32),
            pltpu.VMEM((2, _CHUNK, d), jnp.float32),
            pltpu.VMEM_SHARED((npad, d), jnp.float32),
            pltpu.SemaphoreType.DMA,
            pltpu.SemaphoreType.DMA,
            pltpu.SemaphoreType.DMA,
            pltpu.SemaphoreType.DMA,
        ],
    )
    def aggregate(h_hbm, src_hbm, dst_hbm, out_hbm,
                  src_v, dst_v, rows_v, acc_sh, sem0, sem1, ssem0, ssem1):
        cid = lax.axis_index("c")
        sid = lax.axis_index("s")
        wid = sid * _NC + cid

        z16 = jnp.zeros((16,), jnp.float32)

        def zrow(r, _):
            def zcol(c, _):
                rows_v[0, r, pl.ds(c * 16, 16)] = z16
                return 0
            return lax.fori_loop(0, d // 16, zcol, 0)

        lax.fori_loop(0, _CHUNK, zrow, 0)
        zbase = sid * zrows
        for z in range(nzc):
            pltpu.sync_copy(rows_v.at[0],
                            acc_sh.at[pl.ds(zbase + z * _CHUNK, _CHUNK)])
        if zrem:
            pltpu.sync_copy(rows_v.at[0, pl.ds(0, zrem)],
                            acc_sh.at[pl.ds(zbase + nzc * _CHUNK, zrem)])
        plsc.subcore_barrier()

        for ps in range(npass):
            pltpu.sync_copy(src_hbm.at[wid, pl.ds(ps * kp, kp)], src_v)
            pltpu.sync_copy(dst_hbm.at[wid, pl.ds(ps * kp, kp)], dst_v)

            pltpu.async_copy(h_hbm.at[src_v.at[0]], rows_v.at[0], sem0)
            pltpu.async_copy(h_hbm.at[src_v.at[1]], rows_v.at[1], sem1)

            def pair(p, _):
                a = p * 2
                pltpu.make_async_copy(h_hbm.at[src_v.at[a]], rows_v.at[0],
                                      sem0).wait()
                pltpu.async_copy(rows_v.at[0], acc_sh.at[dst_v.at[a]],
                                 ssem0, add=True)
                pltpu.make_async_copy(h_hbm.at[src_v.at[a + 1]], rows_v.at[1],
                                      sem1).wait()
                pltpu.async_copy(rows_v.at[1], acc_sh.at[dst_v.at[a + 1]],
                                 ssem1, add=True)
                pltpu.make_async_copy(rows_v.at[0], acc_sh.at[dst_v.at[a]],
                                      ssem0).wait()

                @pl.when(a + 2 < kp)
                def _():
                    pltpu.async_copy(h_hbm.at[src_v.at[a + 2]], rows_v.at[0],
                                     sem0)

                pltpu.make_async_copy(rows_v.at[1], acc_sh.at[dst_v.at[a + 1]],
                                      ssem1).wait()

                @pl.when(a + 3 < kp)
                def _():
                    pltpu.async_copy(h_hbm.at[src_v.at[a + 3]], rows_v.at[1],
                                     sem1)

                return 0

            lax.fori_loop(0, kp // 2, pair, 0)

        plsc.subcore_barrier()
        pltpu.sync_copy(acc_sh.at[pl.ds(zbase, zrows)],
                        out_hbm.at[cid, pl.ds(zbase, zrows)])

    return aggregate


@functools.lru_cache(maxsize=None)
def _make_mlp(n, d, h, relu_out, br, npad):
    def body(x_ref, p_ref, w1_ref, b1_ref, w2_ref, b2_ref, o_ref):
        t = x_ref[...] + p_ref[0] + p_ref[1]
        a = jnp.dot(t, w1_ref[...], preferred_element_type=jnp.float32)
        a = jnp.maximum(a + b1_ref[...], 0.0)
        o = jnp.dot(a, w2_ref[...], preferred_element_type=jnp.float32)
        o = o + b2_ref[...]
        if relu_out:
            o = jnp.maximum(o, 0.0)
        o_ref[...] = o

    return pl.pallas_call(
        body,
        grid=(n // br,),
        in_specs=[
            pl.BlockSpec((br, d), lambda i: (i, 0)),
            pl.BlockSpec((2, br, d), lambda i: (0, i, 0)),
            pl.BlockSpec((d, h), lambda i: (0, 0)),
            pl.BlockSpec((1, h), lambda i: (0, 0)),
            pl.BlockSpec((h, h), lambda i: (0, 0)),
            pl.BlockSpec((1, h), lambda i: (0, 0)),
        ],
        out_specs=pl.BlockSpec((br, h), lambda i: (i, 0)),
        out_shape=jax.ShapeDtypeStruct((n, h), jnp.float32),
    )


def kernel(x, edge_index, W1, b1, W2, b2):
    n, d = x.shape
    h = W1.shape[1]
    e = edge_index.shape[1]

    epw = _round_up(-(-e // _NW), 2 * _CHUNK)
    e_pad = epw * _NW
    k = epw // _CHUNK
    kp = next(c for c in (40, 32, 24, 16, 8, k) if c % 8 == 0 and k % c == 0)
    npad = _round_up(n + 1, _CHUNK)
    src = edge_index[0]
    dst = edge_index[1]
    pad = e_pad - e
    if pad:
        pidx = jnp.arange(pad, dtype=jnp.int32)
        src = jnp.concatenate([src, pidx % n])
        dst = jnp.concatenate([dst, n + pidx % (npad - n)])
    srcr = src.reshape(_NW, k, _CHUNK)
    dstr = dst.reshape(_NW, k, _CHUNK)

    aggregate = _make_aggregate(n, d, k, npad, kp)
    br = 1000 if n % 1000 == 0 else n
    mlp_relu = _make_mlp(n, d, h, True, br, npad)
    mlp_lin = _make_mlp(n, d, h, False, br, npad)
    b1r = b1.reshape(1, h)
    b2r = b2.reshape(1, h)

    p1 = aggregate(x, srcr, dstr)
    h1 = mlp_relu(x, p1, W1, b1r, W2, b2r)
    p2 = aggregate(h1, srcr, dstr)
    out = mlp_lin(h1, p2, W1, b1r, W2, b2r)
    return out


# speedup vs baseline: 1.2443x; 1.2443x over previous
import functools

import jax
import jax.numpy as jnp
from jax import lax
from jax.experimental import pallas as pl
from jax.experimental.pallas import tpu as pltpu
from jax.experimental.pallas import tpu_sc as plsc

_NC = 2
_NS = 16
_NW = _NC * _NS
_CHUNK = 128


def _round_up(a: int, b: int) -> int:
    return (a + b - 1) // b * b


@functools.lru_cache(maxsize=None)
def _make_aggregate(n, d, k, npad, kp):
    mesh = plsc.VectorSubcoreMesh(core_axis_name="c", subcore_axis_name="s")
    zrows = npad // _NS
    nzc = zrows // _CHUNK
    zrem = zrows - nzc * _CHUNK
    npass = k // kp

    @functools.partial(
        pl.kernel,
        out_type=jax.ShapeDtypeStruct((_NC, npad, d), jnp.float32),
        mesh=mesh,
        scratch_types=[
            pltpu.VMEM((kp, _CHUNK), jnp.int32),
            pltpu.VMEM((kp, _CHUNK), jnp.int32),
            pltpu.VMEM((2, _CHUNK, d), jnp.float32),
            pltpu.VMEM_SHARED((npad, d), jnp.float32),
            pltpu.SemaphoreType.DMA,
            pltpu.SemaphoreType.DMA,
            pltpu.SemaphoreType.DMA,
            pltpu.SemaphoreType.DMA,
        ],
    )
    def aggregate(h_hbm, src_hbm, dst_hbm, out_hbm,
                  src_v, dst_v, rows_v, acc_sh, sem0, sem1, ssem0, ssem1):
        cid = lax.axis_index("c")
        sid = lax.axis_index("s")
        wid = sid * _NC + cid

        z16 = jnp.zeros((16,), jnp.float32)

        def zrow(r, _):
            def zcol(c, _):
                rows_v[0, r, pl.ds(c * 16, 16)] = z16
                return 0
            return lax.fori_loop(0, d // 16, zcol, 0)

        lax.fori_loop(0, _CHUNK, zrow, 0)
        zbase = sid * zrows
        for z in range(nzc):
            pltpu.sync_copy(rows_v.at[0],
                            acc_sh.at[pl.ds(zbase + z * _CHUNK, _CHUNK)])
        if zrem:
            pltpu.sync_copy(rows_v.at[0, pl.ds(0, zrem)],
                            acc_sh.at[pl.ds(zbase + nzc * _CHUNK, zrem)])
        plsc.subcore_barrier()

        for ps in range(npass):
            pltpu.sync_copy(src_hbm.at[wid, pl.ds(ps * kp, kp)], src_v)
            pltpu.sync_copy(dst_hbm.at[wid, pl.ds(ps * kp, kp)], dst_v)

            hh = _CHUNK // 2

            def fire(j, buf, s0, s1):
                pltpu.async_copy(h_hbm.at[src_v.at[j, pl.ds(0, hh)]],
                                 rows_v.at[buf, pl.ds(0, hh)], s0)
                pltpu.async_copy(h_hbm.at[src_v.at[j, pl.ds(hh, hh)]],
                                 rows_v.at[buf, pl.ds(hh, hh)], s1)

            def drain(j, buf, s0, s1):
                pltpu.make_async_copy(h_hbm.at[src_v.at[j, pl.ds(0, hh)]],
                                      rows_v.at[buf, pl.ds(0, hh)], s0).wait()
                pltpu.make_async_copy(h_hbm.at[src_v.at[j, pl.ds(hh, hh)]],
                                      rows_v.at[buf, pl.ds(hh, hh)], s1).wait()

            fire(0, 0, sem0, ssem0)
            fire(1, 1, sem1, ssem1)

            def pair(p, _):
                a = p * 2
                drain(a, 0, sem0, ssem0)
                pltpu.sync_copy(rows_v.at[0], acc_sh.at[dst_v.at[a]],
                                add=True)

                @pl.when(a + 2 < kp)
                def _():
                    fire(a + 2, 0, sem0, ssem0)

                drain(a + 1, 1, sem1, ssem1)
                pltpu.sync_copy(rows_v.at[1], acc_sh.at[dst_v.at[a + 1]],
                                add=True)

                @pl.when(a + 3 < kp)
                def _():
                    fire(a + 3, 1, sem1, ssem1)

                return 0

            lax.fori_loop(0, kp // 2, pair, 0)

        plsc.subcore_barrier()
        pltpu.sync_copy(acc_sh.at[pl.ds(zbase, zrows)],
                        out_hbm.at[cid, pl.ds(zbase, zrows)])

    return aggregate


@functools.lru_cache(maxsize=None)
def _make_mlp(n, d, h, relu_out, br, npad):
    def body(x_ref, p_ref, w1_ref, b1_ref, w2_ref, b2_ref, o_ref):
        t = x_ref[...] + p_ref[0] + p_ref[1]
        a = jnp.dot(t, w1_ref[...], preferred_element_type=jnp.float32)
        a = jnp.maximum(a + b1_ref[...], 0.0)
        o = jnp.dot(a, w2_ref[...], preferred_element_type=jnp.float32)
        o = o + b2_ref[...]
        if relu_out:
            o = jnp.maximum(o, 0.0)
        o_ref[...] = o

    return pl.pallas_call(
        body,
        grid=(n // br,),
        in_specs=[
            pl.BlockSpec((br, d), lambda i: (i, 0)),
            pl.BlockSpec((2, br, d), lambda i: (0, i, 0)),
            pl.BlockSpec((d, h), lambda i: (0, 0)),
            pl.BlockSpec((1, h), lambda i: (0, 0)),
            pl.BlockSpec((h, h), lambda i: (0, 0)),
            pl.BlockSpec((1, h), lambda i: (0, 0)),
        ],
        out_specs=pl.BlockSpec((br, h), lambda i: (i, 0)),
        out_shape=jax.ShapeDtypeStruct((n, h), jnp.float32),
    )


def kernel(x, edge_index, W1, b1, W2, b2):
    n, d = x.shape
    h = W1.shape[1]
    e = edge_index.shape[1]

    epw = _round_up(-(-e // _NW), 2 * _CHUNK)
    e_pad = epw * _NW
    k = epw // _CHUNK
    kp = next(c for c in (40, 32, 24, 16, 8, k) if c % 8 == 0 and k % c == 0)
    npad = _round_up(n + 1, _CHUNK)
    src = edge_index[0]
    dst = edge_index[1]
    pad = e_pad - e
    if pad:
        pidx = jnp.arange(pad, dtype=jnp.int32)
        src = jnp.concatenate([src, pidx % n])
        dst = jnp.concatenate([dst, n + pidx % (npad - n)])
    srcr = src.reshape(_NW, k, _CHUNK)
    dstr = dst.reshape(_NW, k, _CHUNK)

    aggregate = _make_aggregate(n, d, k, npad, kp)
    br = 1000 if n % 1000 == 0 else n
    mlp_relu = _make_mlp(n, d, h, True, br, npad)
    mlp_lin = _make_mlp(n, d, h, False, br, npad)
    b1r = b1.reshape(1, h)
    b2r = b2.reshape(1, h)

    p1 = aggregate(x, srcr, dstr)
    h1 = mlp_relu(x, p1, W1, b1r, W2, b2r)
    p2 = aggregate(h1, srcr, dstr)
    out = mlp_lin(h1, p2, W1, b1r, W2, b2r)
    return out


# speedup vs baseline: 1.2624x; 1.0146x over previous
import functools

import jax
import jax.numpy as jnp
from jax import lax
from jax.experimental import pallas as pl
from jax.experimental.pallas import tpu as pltpu
from jax.experimental.pallas import tpu_sc as plsc

_NC = 2
_NS = 16
_NW = _NC * _NS
_CHUNK = 128


def _round_up(a: int, b: int) -> int:
    return (a + b - 1) // b * b


@functools.lru_cache(maxsize=None)
def _make_aggregate(n, d, k, npad, kp):
    mesh = plsc.VectorSubcoreMesh(core_axis_name="c", subcore_axis_name="s")
    zrows = npad // _NS
    nzc = zrows // _CHUNK
    zrem = zrows - nzc * _CHUNK
    npass = k // kp

    @functools.partial(
        pl.kernel,
        out_type=jax.ShapeDtypeStruct((_NC, npad, d), jnp.float32),
        mesh=mesh,
        scratch_types=[
            pltpu.VMEM((kp, _CHUNK), jnp.int32),
            pltpu.VMEM((kp, _CHUNK), jnp.int32),
            pltpu.VMEM((2, _CHUNK, d), jnp.float32),
            pltpu.VMEM_SHARED((npad, d), jnp.float32),
            pltpu.SemaphoreType.DMA,
            pltpu.SemaphoreType.DMA,
            pltpu.SemaphoreType.DMA,
            pltpu.SemaphoreType.DMA,
        ],
    )
    def aggregate(h_hbm, src_hbm, dst_hbm, out_hbm,
                  src_v, dst_v, rows_v, acc_sh, sem0, sem1, ssem0, ssem1):
        cid = lax.axis_index("c")
        sid = lax.axis_index("s")
        wid = sid * _NC + cid

        z16 = jnp.zeros((16,), jnp.float32)

        def zrow(r, _):
            def zcol(c, _):
                rows_v[0, r, pl.ds(c * 16, 16)] = z16
                return 0
            return lax.fori_loop(0, d // 16, zcol, 0)

        lax.fori_loop(0, _CHUNK, zrow, 0)
        zbase = sid * zrows
        for z in range(nzc):
            pltpu.sync_copy(rows_v.at[0],
                            acc_sh.at[pl.ds(zbase + z * _CHUNK, _CHUNK)])
        if zrem:
            pltpu.sync_copy(rows_v.at[0, pl.ds(0, zrem)],
                            acc_sh.at[pl.ds(zbase + nzc * _CHUNK, zrem)])
        plsc.subcore_barrier()

        for ps in range(npass):
            pltpu.sync_copy(src_hbm.at[wid, pl.ds(ps * kp, kp)], src_v)
            pltpu.sync_copy(dst_hbm.at[wid, pl.ds(ps * kp, kp)], dst_v)

            pltpu.async_copy(h_hbm.at[src_v.at[0]], rows_v.at[0], sem0)
            pltpu.async_copy(h_hbm.at[src_v.at[1]], rows_v.at[1], sem1)

            hh = _CHUNK // 2

            def scatter(j, buf):
                pltpu.async_copy(rows_v.at[buf, pl.ds(0, hh)],
                                 acc_sh.at[dst_v.at[j, pl.ds(0, hh)]],
                                 ssem0, add=True)
                pltpu.async_copy(rows_v.at[buf, pl.ds(hh, hh)],
                                 acc_sh.at[dst_v.at[j, pl.ds(hh, hh)]],
                                 ssem1, add=True)
                pltpu.make_async_copy(rows_v.at[buf, pl.ds(0, hh)],
                                      acc_sh.at[dst_v.at[j, pl.ds(0, hh)]],
                                      ssem0).wait()
                pltpu.make_async_copy(rows_v.at[buf, pl.ds(hh, hh)],
                                      acc_sh.at[dst_v.at[j, pl.ds(hh, hh)]],
                                      ssem1).wait()

            def pair(p, _):
                a = p * 2
                pltpu.make_async_copy(h_hbm.at[src_v.at[a]], rows_v.at[0],
                                      sem0).wait()
                scatter(a, 0)

                @pl.when(a + 2 < kp)
                def _():
                    pltpu.async_copy(h_hbm.at[src_v.at[a + 2]], rows_v.at[0],
                                     sem0)

                pltpu.make_async_copy(h_hbm.at[src_v.at[a + 1]], rows_v.at[1],
                                      sem1).wait()
                scatter(a + 1, 1)

                @pl.when(a + 3 < kp)
                def _():
                    pltpu.async_copy(h_hbm.at[src_v.at[a + 3]], rows_v.at[1],
                                     sem1)

                return 0

            lax.fori_loop(0, kp // 2, pair, 0)

        plsc.subcore_barrier()
        pltpu.sync_copy(acc_sh.at[pl.ds(zbase, zrows)],
                        out_hbm.at[cid, pl.ds(zbase, zrows)])

    return aggregate


@functools.lru_cache(maxsize=None)
def _make_mlp(n, d, h, relu_out, br, npad):
    def body(x_ref, p_ref, w1_ref, b1_ref, w2_ref, b2_ref, o_ref):
        t = x_ref[...] + p_ref[0] + p_ref[1]
        a = jnp.dot(t, w1_ref[...], preferred_element_type=jnp.float32)
        a = jnp.maximum(a + b1_ref[...], 0.0)
        o = jnp.dot(a, w2_ref[...], preferred_element_type=jnp.float32)
        o = o + b2_ref[...]
        if relu_out:
            o = jnp.maximum(o, 0.0)
        o_ref[...] = o

    return pl.pallas_call(
        body,
        grid=(n // br,),
        in_specs=[
            pl.BlockSpec((br, d), lambda i: (i, 0)),
            pl.BlockSpec((2, br, d), lambda i: (0, i, 0)),
            pl.BlockSpec((d, h), lambda i: (0, 0)),
            pl.BlockSpec((1, h), lambda i: (0, 0)),
            pl.BlockSpec((h, h), lambda i: (0, 0)),
            pl.BlockSpec((1, h), lambda i: (0, 0)),
        ],
        out_specs=pl.BlockSpec((br, h), lambda i: (i, 0)),
        out_shape=jax.ShapeDtypeStruct((n, h), jnp.float32),
    )


def kernel(x, edge_index, W1, b1, W2, b2):
    n, d = x.shape
    h = W1.shape[1]
    e = edge_index.shape[1]

    epw = _round_up(-(-e // _NW), 2 * _CHUNK)
    e_pad = epw * _NW
    k = epw // _CHUNK
    kp = next(c for c in (40, 32, 24, 16, 8, k) if c % 8 == 0 and k % c == 0)
    npad = _round_up(n + 1, _CHUNK)
    src = edge_index[0]
    dst = edge_index[1]
    pad = e_pad - e
    if pad:
        pidx = jnp.arange(pad, dtype=jnp.int32)
        src = jnp.concatenate([src, pidx % n])
        dst = jnp.concatenate([dst, n + pidx % (npad - n)])
    srcr = src.reshape(_NW, k, _CHUNK)
    dstr = dst.reshape(_NW, k, _CHUNK)

    aggregate = _make_aggregate(n, d, k, npad, kp)
    br = 1000 if n % 1000 == 0 else n
    mlp_relu = _make_mlp(n, d, h, True, br, npad)
    mlp_lin = _make_mlp(n, d, h, False, br, npad)
    b1r = b1.reshape(1, h)
    b2r = b2.reshape(1, h)

    p1 = aggregate(x, srcr, dstr)
    h1 = mlp_relu(x, p1, W1, b1r, W2, b2r)
    p2 = aggregate(h1, srcr, dstr)
    out = mlp_lin(h1, p2, W1, b1r, W2, b2r)
    return out


# speedup vs baseline: 1.2962x; 1.0268x over previous
import functools

import jax
import jax.numpy as jnp
from jax import lax
from jax.experimental import pallas as pl
from jax.experimental.pallas import tpu as pltpu
from jax.experimental.pallas import tpu_sc as plsc

_NC = 2
_NS = 16
_NW = _NC * _NS
_CHUNK = 128


def _round_up(a: int, b: int) -> int:
    return (a + b - 1) // b * b


@functools.lru_cache(maxsize=None)
def _make_aggregate(n, d, k, npad, kp):
    mesh = plsc.VectorSubcoreMesh(core_axis_name="c", subcore_axis_name="s")
    zrows = npad // _NS
    nzc = zrows // _CHUNK
    zrem = zrows - nzc * _CHUNK
    npass = k // kp

    @functools.partial(
        pl.kernel,
        out_type=jax.ShapeDtypeStruct((_NC, npad, d), jnp.float32),
        mesh=mesh,
        scratch_types=[
            pltpu.VMEM((kp, _CHUNK), jnp.int32),
            pltpu.VMEM((kp, _CHUNK), jnp.int32),
            pltpu.VMEM((2, _CHUNK, d), jnp.float32),
            pltpu.VMEM_SHARED((npad, d), jnp.float32),
            pltpu.SemaphoreType.DMA,
            pltpu.SemaphoreType.DMA,
            pltpu.SemaphoreType.DMA,
            pltpu.SemaphoreType.DMA,
        ],
    )
    def aggregate(h_hbm, src_hbm, dst_hbm, out_hbm,
                  src_v, dst_v, rows_v, acc_sh, sem0, sem1, ssem0, ssem1):
        cid = lax.axis_index("c")
        sid = lax.axis_index("s")
        wid = sid * _NC + cid

        z16 = jnp.zeros((16,), jnp.float32)

        def zrow(r, _):
            def zcol(c, _):
                rows_v[0, r, pl.ds(c * 16, 16)] = z16
                return 0
            return lax.fori_loop(0, d // 16, zcol, 0)

        lax.fori_loop(0, _CHUNK, zrow, 0)
        zbase = sid * zrows
        for z in range(nzc):
            pltpu.sync_copy(rows_v.at[0],
                            acc_sh.at[pl.ds(zbase + z * _CHUNK, _CHUNK)])
        if zrem:
            pltpu.sync_copy(rows_v.at[0, pl.ds(0, zrem)],
                            acc_sh.at[pl.ds(zbase + nzc * _CHUNK, zrem)])
        plsc.subcore_barrier()

        for ps in range(npass):
            pltpu.sync_copy(src_hbm.at[wid, pl.ds(ps * kp, kp)], src_v)
            pltpu.sync_copy(dst_hbm.at[wid, pl.ds(ps * kp, kp)], dst_v)

            pltpu.async_copy(h_hbm.at[src_v.at[0]], rows_v.at[0], sem0)
            pltpu.async_copy(h_hbm.at[src_v.at[1]], rows_v.at[1], sem1)

            hh = _CHUNK // 2

            def scatter(j, buf):
                pltpu.async_copy(rows_v.at[buf, pl.ds(0, hh)],
                                 acc_sh.at[dst_v.at[j, pl.ds(0, hh)]],
                                 ssem0, add=True)
                pltpu.async_copy(rows_v.at[buf, pl.ds(hh, hh)],
                                 acc_sh.at[dst_v.at[j, pl.ds(hh, hh)]],
                                 ssem1, add=True)
                pltpu.make_async_copy(rows_v.at[buf, pl.ds(0, hh)],
                                      acc_sh.at[dst_v.at[j, pl.ds(0, hh)]],
                                      ssem0).wait()
                pltpu.make_async_copy(rows_v.at[buf, pl.ds(hh, hh)],
                                      acc_sh.at[dst_v.at[j, pl.ds(hh, hh)]],
                                      ssem1).wait()

            def pair(p, _):
                a = p * 2
                pltpu.make_async_copy(h_hbm.at[src_v.at[a]], rows_v.at[0],
                                      sem0).wait()
                scatter(a, 0)

                @pl.when(a + 2 < kp)
                def _():
                    pltpu.async_copy(h_hbm.at[src_v.at[a + 2]], rows_v.at[0],
                                     sem0)

                pltpu.make_async_copy(h_hbm.at[src_v.at[a + 1]], rows_v.at[1],
                                      sem1).wait()
                scatter(a + 1, 1)

                @pl.when(a + 3 < kp)
                def _():
                    pltpu.async_copy(h_hbm.at[src_v.at[a + 3]], rows_v.at[1],
                                     sem1)

                return 0

            lax.fori_loop(0, kp // 2, pair, 0)

        plsc.subcore_barrier()
        pltpu.sync_copy(acc_sh.at[pl.ds(zbase, zrows)],
                        out_hbm.at[cid, pl.ds(zbase, zrows)])

    return aggregate


@functools.lru_cache(maxsize=None)
def _make_mlp(n, d, h, relu_out, br, npad):
    def body(x_ref, p_ref, w1_ref, b1_ref, w2_ref, b2_ref, o_ref):
        t = x_ref[...] + p_ref[0] + p_ref[1]
        a = jnp.dot(t, w1_ref[...], preferred_element_type=jnp.float32)
        a = jnp.maximum(a + b1_ref[...], 0.0)
        o = jnp.dot(a, w2_ref[...], preferred_element_type=jnp.float32)
        o = o + b2_ref[...]
        if relu_out:
            o = jnp.maximum(o, 0.0)
        o_ref[...] = o

    return pl.pallas_call(
        body,
        grid=(n // br,),
        in_specs=[
            pl.BlockSpec((br, d), lambda i: (i, 0)),
            pl.BlockSpec((2, br, d), lambda i: (0, i, 0)),
            pl.BlockSpec((d, h), lambda i: (0, 0)),
            pl.BlockSpec((1, h), lambda i: (0, 0)),
            pl.BlockSpec((h, h), lambda i: (0, 0)),
            pl.BlockSpec((1, h), lambda i: (0, 0)),
        ],
        out_specs=pl.BlockSpec((br, h), lambda i: (i, 0)),
        out_shape=jax.ShapeDtypeStruct((n, h), jnp.float32),
    )


def kernel(x, edge_index, W1, b1, W2, b2):
    n, d = x.shape
    h = W1.shape[1]
    e = edge_index.shape[1]

    epw = _round_up(-(-e // _NW), 2 * _CHUNK)
    e_pad = epw * _NW
    k = epw // _CHUNK
    kp = next(c for c in (40, 32, 24, 16, 8, k) if c % 8 == 0 and k % c == 0)
    npad = _round_up(n + 1, _CHUNK)
    src = edge_index[0]
    dst = edge_index[1]
    pad = e_pad - e
    if pad:
        pidx = jnp.arange(pad, dtype=jnp.int32)
        src = jnp.concatenate([src, pidx % n])
        dst = jnp.concatenate([dst, n + pidx % (npad - n)])
    srcr = src.reshape(_NW, k, _CHUNK)
    dstr = dst.reshape(_NW, k, _CHUNK)

    aggregate = _make_aggregate(n, d, k, npad, kp)
    br = 2000 if n % 2000 == 0 else (1000 if n % 1000 == 0 else n)
    mlp_relu = _make_mlp(n, d, h, True, br, npad)
    mlp_lin = _make_mlp(n, d, h, False, br, npad)
    b1r = b1.reshape(1, h)
    b2r = b2.reshape(1, h)

    p1 = aggregate(x, srcr, dstr)
    h1 = mlp_relu(x, p1, W1, b1r, W2, b2r)
    p2 = aggregate(h1, srcr, dstr)
    out = mlp_lin(h1, p2, W1, b1r, W2, b2r)
    return out


# speedup vs baseline: 1.3068x; 1.0082x over previous
import functools

import jax
import jax.numpy as jnp
from jax import lax
from jax.experimental import pallas as pl
from jax.experimental.pallas import tpu as pltpu
from jax.experimental.pallas import tpu_sc as plsc

_NC = 2
_NS = 16
_NW = _NC * _NS
_CHUNK = 128


def _round_up(a: int, b: int) -> int:
    return (a + b - 1) // b * b


@functools.lru_cache(maxsize=None)
def _make_aggregate(n, d, k, npad, kp):
    mesh = plsc.VectorSubcoreMesh(core_axis_name="c", subcore_axis_name="s")
    zrows = npad // _NS
    nzc = zrows // _CHUNK
    zrem = zrows - nzc * _CHUNK
    npass = k // kp

    @functools.partial(
        pl.kernel,
        out_type=jax.ShapeDtypeStruct((_NC, npad, d), jnp.float32),
        mesh=mesh,
        scratch_types=[
            pltpu.VMEM((kp, _CHUNK), jnp.int32),
            pltpu.VMEM((kp, _CHUNK), jnp.int32),
            pltpu.VMEM((2, _CHUNK, d), jnp.float32),
            pltpu.VMEM_SHARED((npad, d), jnp.float32),
            pltpu.SemaphoreType.DMA,
            pltpu.SemaphoreType.DMA,
            pltpu.SemaphoreType.DMA,
            pltpu.SemaphoreType.DMA,
        ],
    )
    def aggregate(h_hbm, src_hbm, dst_hbm, out_hbm,
                  src_v, dst_v, rows_v, acc_sh, sem0, sem1, ssem0, ssem1):
        cid = lax.axis_index("c")
        sid = lax.axis_index("s")
        wid = sid * _NC + cid

        z16 = jnp.zeros((16,), jnp.float32)

        def zrow(r, _):
            def zcol(c, _):
                rows_v[0, r, pl.ds(c * 16, 16)] = z16
                return 0
            return lax.fori_loop(0, d // 16, zcol, 0)

        lax.fori_loop(0, _CHUNK, zrow, 0)
        zbase = sid * zrows
        for z in range(nzc):
            pltpu.sync_copy(rows_v.at[0],
                            acc_sh.at[pl.ds(zbase + z * _CHUNK, _CHUNK)])
        if zrem:
            pltpu.sync_copy(rows_v.at[0, pl.ds(0, zrem)],
                            acc_sh.at[pl.ds(zbase + nzc * _CHUNK, zrem)])
        plsc.subcore_barrier()

        for ps in range(npass):
            pltpu.sync_copy(src_hbm.at[wid, pl.ds(ps * kp, kp)], src_v)
            pltpu.sync_copy(dst_hbm.at[wid, pl.ds(ps * kp, kp)], dst_v)

            pltpu.async_copy(h_hbm.at[src_v.at[0]], rows_v.at[0], sem0)
            pltpu.async_copy(h_hbm.at[src_v.at[1]], rows_v.at[1], sem1)

            hh = _CHUNK // 2

            def scatter(j, buf):
                pltpu.async_copy(rows_v.at[buf, pl.ds(0, hh)],
                                 acc_sh.at[dst_v.at[j, pl.ds(0, hh)]],
                                 ssem0, add=True)
                pltpu.async_copy(rows_v.at[buf, pl.ds(hh, hh)],
                                 acc_sh.at[dst_v.at[j, pl.ds(hh, hh)]],
                                 ssem1, add=True)
                pltpu.make_async_copy(rows_v.at[buf, pl.ds(0, hh)],
                                      acc_sh.at[dst_v.at[j, pl.ds(0, hh)]],
                                      ssem0).wait()
                pltpu.make_async_copy(rows_v.at[buf, pl.ds(hh, hh)],
                                      acc_sh.at[dst_v.at[j, pl.ds(hh, hh)]],
                                      ssem1).wait()

            def pair(p, _):
                a = p * 2
                pltpu.make_async_copy(h_hbm.at[src_v.at[a]], rows_v.at[0],
                                      sem0).wait()
                scatter(a, 0)

                @pl.when(a + 2 < kp)
                def _():
                    pltpu.async_copy(h_hbm.at[src_v.at[a + 2]], rows_v.at[0],
                                     sem0)

                pltpu.make_async_copy(h_hbm.at[src_v.at[a + 1]], rows_v.at[1],
                                      sem1).wait()
                scatter(a + 1, 1)

                @pl.when(a + 3 < kp)
                def _():
                    pltpu.async_copy(h_hbm.at[src_v.at[a + 3]], rows_v.at[1],
                                     sem1)

                return 0

            lax.fori_loop(0, kp // 2, pair, 0)

        plsc.subcore_barrier()
        pltpu.sync_copy(acc_sh.at[pl.ds(zbase, zrows)],
                        out_hbm.at[cid, pl.ds(zbase, zrows)])

    return aggregate


@functools.lru_cache(maxsize=None)
def _make_mlp(n, d, h, relu_out, br, npad):
    def body(x_ref, p_ref, w1_ref, b1_ref, w2_ref, b2_ref, o_ref):
        t = x_ref[...] + p_ref[0] + p_ref[1]
        a = jnp.dot(t, w1_ref[...], preferred_element_type=jnp.float32)
        a = jnp.maximum(a + b1_ref[...], 0.0)
        o = jnp.dot(a, w2_ref[...], preferred_element_type=jnp.float32)
        o = o + b2_ref[...]
        if relu_out:
            o = jnp.maximum(o, 0.0)
        o_ref[...] = o

    return pl.pallas_call(
        body,
        grid=(n // br,),
        in_specs=[
            pl.BlockSpec((br, d), lambda i: (i, 0)),
            pl.BlockSpec((2, br, d), lambda i: (0, i, 0)),
            pl.BlockSpec((d, h), lambda i: (0, 0)),
            pl.BlockSpec((1, h), lambda i: (0, 0)),
            pl.BlockSpec((h, h), lambda i: (0, 0)),
            pl.BlockSpec((1, h), lambda i: (0, 0)),
        ],
        out_specs=pl.BlockSpec((br, h), lambda i: (i, 0)),
        out_shape=jax.ShapeDtypeStruct((n, h), jnp.float32),
    )


def kernel(x, edge_index, W1, b1, W2, b2):
    n, d = x.shape
    h = W1.shape[1]
    e = edge_index.shape[1]

    epw = _round_up(-(-e // _NW), 2 * _CHUNK)
    e_pad = epw * _NW
    k = epw // _CHUNK
    kp = next(c for c in (40, 32, 24, 16, 8, k) if c % 8 == 0 and k % c == 0)
    npad = _round_up(n + 1, _CHUNK)
    src = edge_index[0]
    dst = edge_index[1]
    pad = e_pad - e
    if pad:
        pidx = jnp.arange(pad, dtype=jnp.int32)
        src = jnp.concatenate([src, pidx % n])
        dst = jnp.concatenate([dst, n + pidx % (npad - n)])
    srcr = src.reshape(_NW, k, _CHUNK)
    dstr = dst.reshape(_NW, k, _CHUNK)

    aggregate = _make_aggregate(n, d, k, npad, kp)
    br = 5000 if n % 5000 == 0 else (1000 if n % 1000 == 0 else n)
    mlp_relu = _make_mlp(n, d, h, True, br, npad)
    mlp_lin = _make_mlp(n, d, h, False, br, npad)
    b1r = b1.reshape(1, h)
    b2r = b2.reshape(1, h)

    p1 = aggregate(x, srcr, dstr)
    h1 = mlp_relu(x, p1, W1, b1r, W2, b2r)
    p2 = aggregate(h1, srcr, dstr)
    out = mlp_lin(h1, p2, W1, b1r, W2, b2r)
    return out
